# baseline (speedup 1.0000x reference)
import functools

import jax
import jax.numpy as jnp
from jax import lax
from jax.experimental import pallas as pl
from jax.experimental.pallas import tpu as pltpu
from jax.experimental.pallas import tpu_sc as plsc

_CHUNK = 128


@functools.lru_cache(maxsize=None)
def _make_gather(V, D, B):
    info = plsc.get_sparse_core_info()
    NC, NS, L = info.num_cores, info.num_subcores, info.num_lanes
    NW = NC * NS
    assert B % (NW * _CHUNK) == 0 and D % L == 0 and V % 2 == 0
    b_per_w = B // NW
    n_waves = b_per_w // _CHUNK
    nvec = D // L
    mesh = plsc.VectorSubcoreMesh(core_axis_name="c", subcore_axis_name="s")

    @functools.partial(
        pl.kernel,
        mesh=mesh,
        out_type=jax.ShapeDtypeStruct((B, D), jnp.float32),
        scratch_types=[
            pltpu.VMEM((b_per_w,), jnp.int32),
            pltpu.VMEM((b_per_w,), jnp.int32),
            pltpu.VMEM((2, _CHUNK, 2 * D), jnp.float32),
            pltpu.VMEM((b_per_w, D), jnp.float32),
            [pltpu.SemaphoreType.DMA] * 2,
        ],
    )
    def gather_kernel(nodes_hbm, pairs_hbm, out_hbm, idx_v, pid_v, bufs_v,
                      rows_v, sems):
        wid = lax.axis_index("s") * NC + lax.axis_index("c")
        base = wid * b_per_w
        pltpu.sync_copy(nodes_hbm.at[pl.ds(base, b_per_w)], idx_v)

        def to_pair(g, carry):
            vec = idx_v[pl.ds(g * L, L)]
            pid_v[pl.ds(g * L, L)] = lax.shift_right_logical(vec, 1)
            return carry

        lax.fori_loop(0, b_per_w // L, to_pair, 0)

        def start(w):
            return pltpu.async_copy(
                pairs_hbm.at[pid_v.at[pl.ds(w * _CHUNK, _CHUNK)]],
                bufs_v.at[w % 2],
                sems[w % 2],
            )

        copies = [None] * n_waves
        copies[0] = start(0)
        for w in range(n_waves):
            if w + 1 < n_waves:
                copies[w + 1] = start(w + 1)
            copies[w].wait()
            for g in range(_CHUNK // L):
                vec = idx_v[pl.ds(w * _CHUNK + g * L, L)]
                off = (vec & 1) * D
                for k in range(L):
                    o = off[k]
                    for c in range(nvec):
                        rows_v[w * _CHUNK + g * L + k, pl.ds(c * L, L)] = (
                            bufs_v[w % 2, g * L + k, pl.ds(o + c * L, L)]
                        )
        pltpu.sync_copy(rows_v, out_hbm.at[pl.ds(base, b_per_w)])

    return gather_kernel


def kernel(nodes, table):
    (B,) = nodes.shape
    V, D = table.shape
    pairs = table.reshape(V // 2, 2 * D)
    return _make_gather(V, D, B)(nodes.astype(jnp.int32), pairs)
